# baseline (speedup 1.0000x reference)
import functools

import jax
import jax.numpy as jnp
import numpy as np
from jax import lax
from jax.experimental import pallas as pl
from jax.experimental.pallas import tpu as pltpu
from jax.experimental.pallas import tpu_sc as plsc

N = 50000
E = 800000
HID = 64
H = 4
DK = 16
NC = 2
NS = 16
NW = NC * NS

C = 128
NCHUNK = E // C
GITER = (NCHUNK + NW - 1) // NW
SITER = (NCHUNK + NS - 1) // NS
KS = 3
SSTEP = (SITER + KS - 1) // KS
NZ = 6
ZR = 8336
NACC = NZ * ZR
TROWS = 8344
ZITER = ZR // C // NS + 1
WITER = ZR // C // NS + 1


def _encode_body(x0, we, be, wa, ba, wq, bq, wkv, bkv, x_out, q_out, kv_out):
    h = jnp.maximum(x0[...] @ we[...] + be[...], 0.0)
    x = jnp.tanh(h @ wa[...] + ba[...])
    x_out[...] = x
    q_out[...] = x @ wq[...] + bq[...]
    kv_out[...] = x @ wkv[...] + bkv[...]


def _edge_body(gq, gkv, sp, b8, w_out):
    qk = gq[:, :64] * gkv[:, :64]
    att = jnp.minimum(qk @ sp[...], 75.0)
    e8 = jnp.exp(att)
    vext = jnp.concatenate(
        [gkv[:, 64:], jnp.ones((gkv.shape[0], 64), jnp.float32)], axis=1)
    w_out[...] = (e8 @ b8[...]) * vext


def _post_body(acc, x, b4, wa, ba, oma, g, b, x_out):
    rec4 = 1.0 / (acc[:, 64:68] + 1e-16)
    rec8 = jnp.concatenate(
        [rec4, jnp.zeros((acc.shape[0], 4), jnp.float32)], axis=1)
    aggr = acc[:, :64] * (rec8 @ b4[...])
    out = jax.nn.gelu(aggr)
    res = out @ wa[...] + ba[...] + x[...] * oma[...]
    mu = jnp.mean(res, axis=-1, keepdims=True)
    var = jnp.mean((res - mu) ** 2, axis=-1, keepdims=True)
    x_out[...] = (res - mu) / jnp.sqrt(var + 1e-5) * g[...] + b[...]


def _post_qkv_body(acc, x, b4, wa, ba, oma, g, b, wq, bq, wkv, bkv,
                   x_out, q_out, kv_out):
    rec4 = 1.0 / (acc[:, 64:68] + 1e-16)
    rec8 = jnp.concatenate(
        [rec4, jnp.zeros((acc.shape[0], 4), jnp.float32)], axis=1)
    aggr = acc[:, :64] * (rec8 @ b4[...])
    out = jax.nn.gelu(aggr)
    res = out @ wa[...] + ba[...] + x[...] * oma[...]
    mu = jnp.mean(res, axis=-1, keepdims=True)
    var = jnp.mean((res - mu) ** 2, axis=-1, keepdims=True)
    y = (res - mu) / jnp.sqrt(var + 1e-5) * g[...] + b[...]
    x_out[...] = y
    q_out[...] = y @ wq[...] + bq[...]
    kv_out[...] = y @ wkv[...] + bkv[...]


def _row_spec(bs, w):
    return pl.BlockSpec((bs, w), lambda i: (i, 0))


def _full_spec(r, c):
    return pl.BlockSpec((r, c), lambda i: (0, 0))


def _encode_call(x0, we, be, wa, ba, wq, bq, wkv, bkv):
    bs = 2000
    return pl.pallas_call(
        _encode_body,
        grid=(N // bs,),
        in_specs=[_row_spec(bs, 128), _full_spec(128, 256), _full_spec(1, 256),
                  _full_spec(256, 64), _full_spec(1, 64), _full_spec(64, 128),
                  _full_spec(1, 128), _full_spec(64, 128), _full_spec(1, 128)],
        out_specs=(_row_spec(bs, 64), _row_spec(bs, 128), _row_spec(bs, 128)),
        out_shape=(jax.ShapeDtypeStruct((N, 64), jnp.float32),
                   jax.ShapeDtypeStruct((N, 128), jnp.float32),
                   jax.ShapeDtypeStruct((N, 128), jnp.float32)),
    )(x0, we, be, wa, ba, wq, bq, wkv, bkv)


def _edge_call(gq, gkv, sp, b8):
    bs = 4000
    return pl.pallas_call(
        _edge_body,
        grid=(E // bs,),
        in_specs=[_row_spec(bs, 128), _row_spec(bs, 128), _full_spec(64, 8),
                  _full_spec(8, 128)],
        out_specs=_row_spec(bs, 128),
        out_shape=jax.ShapeDtypeStruct((E, 128), jnp.float32),
    )(gq, gkv, sp, b8)


def _post_call(acc, x, b4, wa, ba, oma, g, b):
    bs = 2000
    return pl.pallas_call(
        _post_body,
        grid=(N // bs,),
        in_specs=[_row_spec(bs, 128), _row_spec(bs, 64), _full_spec(8, 64),
                  _full_spec(64, 64), _full_spec(1, 64), _full_spec(1, 64),
                  _full_spec(1, 64), _full_spec(1, 64)],
        out_specs=_row_spec(bs, 64),
        out_shape=jax.ShapeDtypeStruct((N, 64), jnp.float32),
    )(acc, x, b4, wa, ba, oma, g, b)


def _post_qkv_call(acc, x, b4, wa, ba, oma, g, b, wq, bq, wkv, bkv):
    bs = 2000
    return pl.pallas_call(
        _post_qkv_body,
        grid=(N // bs,),
        in_specs=[_row_spec(bs, 128), _row_spec(bs, 64), _full_spec(8, 64),
                  _full_spec(64, 64), _full_spec(1, 64), _full_spec(1, 64),
                  _full_spec(1, 64), _full_spec(1, 64), _full_spec(64, 128),
                  _full_spec(1, 128), _full_spec(64, 128), _full_spec(1, 128)],
        out_specs=(_row_spec(bs, 64), _row_spec(bs, 128), _row_spec(bs, 128)),
        out_shape=(jax.ShapeDtypeStruct((N, 64), jnp.float32),
                   jax.ShapeDtypeStruct((N, 128), jnp.float32),
                   jax.ShapeDtypeStruct((N, 128), jnp.float32)),
    )(acc, x, b4, wa, ba, oma, g, b, wq, bq, wkv, bkv)


_MESH = plsc.VectorSubcoreMesh(
    core_axis_name="c", subcore_axis_name="s", num_cores=NC, num_subcores=NS)


KG = 3
GSTEP = (GITER + KG - 1) // KG


@functools.partial(
    pl.kernel,
    out_type=(jax.ShapeDtypeStruct((E, 128), jnp.float32),
              jax.ShapeDtypeStruct((E, 128), jnp.float32)),
    mesh=_MESH,
    scratch_types=[
        [pltpu.VMEM((C,), jnp.int32)] * KG,
        [pltpu.VMEM((C,), jnp.int32)] * KG,
        [pltpu.VMEM((C, 128), jnp.float32)] * KG,
        [pltpu.VMEM((C, 128), jnp.float32)] * KG,
        [pltpu.SemaphoreType.DMA] * KG,
        [pltpu.SemaphoreType.DMA] * KG,
        [pltpu.SemaphoreType.DMA] * KG,
    ],
)
def _gather_kernel(qtab, kvtab, src, dst, gq_out, gkv_out,
                   dsti, srci, qrows, kvrows, semi, semq, semk):
    wid = lax.axis_index("s") * NC + lax.axis_index("c")

    def body(s, carry):
        cids = [(s * KG + b) * NW + wid for b in range(KG)]
        for b in range(KG):
            @pl.when(cids[b] < NCHUNK)
            def _(b=b):
                off = cids[b] * C
                pltpu.async_copy(dst.at[pl.ds(off, C)], dsti[b], semi[b])
                pltpu.async_copy(src.at[pl.ds(off, C)], srci[b], semi[b])
        for b in range(KG):
            @pl.when(cids[b] < NCHUNK)
            def _(b=b):
                off = cids[b] * C
                pltpu.make_async_copy(dst.at[pl.ds(off, C)], dsti[b],
                                      semi[b]).wait()
                pltpu.make_async_copy(src.at[pl.ds(off, C)], srci[b],
                                      semi[b]).wait()
                pltpu.async_copy(qtab.at[dsti[b]], qrows[b], semq[b])
                pltpu.async_copy(kvtab.at[srci[b]], kvrows[b], semk[b])
        for b in range(KG):
            @pl.when(cids[b] < NCHUNK)
            def _(b=b):
                off = cids[b] * C
                pltpu.make_async_copy(qtab.at[dsti[b]], qrows[b],
                                      semq[b]).wait()
                pltpu.make_async_copy(kvtab.at[srci[b]], kvrows[b],
                                      semk[b]).wait()
                pltpu.sync_copy(qrows[b], gq_out.at[pl.ds(off, C)])
                pltpu.sync_copy(kvrows[b], gkv_out.at[pl.ds(off, C)])

        return carry

    lax.fori_loop(0, GSTEP, body, 0)


@functools.partial(
    pl.kernel,
    out_type=jax.ShapeDtypeStruct((NACC, 128), jnp.float32),
    mesh=_MESH,
    scratch_types=[
        [pltpu.VMEM((C,), jnp.int32)] * KS,
        [pltpu.VMEM((C,), jnp.int32)] * KS,
        [pltpu.VMEM((C, 128), jnp.float32)] * KS,
        pltpu.VMEM_SHARED((TROWS, 128), jnp.float32),
        [pltpu.SemaphoreType.DMA] * KS,
    ],
)
def _scatter_kernel(wmsg, dst, zin, acc_out, dsti, idxb, wrows, table, semw):
    cidx = lax.axis_index("c")
    tid = lax.axis_index("s")

    for zi in range(NZ // NC):
        zone = cidx * (NZ // NC) + zi
        base_n = zone * ZR

        pltpu.sync_copy(zin, wrows[0])

        def zbody(j, carry):
            rcid = j * NS + tid

            @pl.when(rcid < ZR // C)
            def _():
                pltpu.sync_copy(wrows[0], table.at[pl.ds(rcid * C, C)])

            return carry

        lax.fori_loop(0, ZITER, zbody, 0)
        ztail = ZR - (ZR // C) * C + 8
        if ztail:
            @pl.when(tid == 0)
            def _():
                r = (ZR // C) * C
                pltpu.sync_copy(wrows[0].at[pl.ds(0, ztail)],
                                table.at[pl.ds(r, ztail)])

        plsc.subcore_barrier()

        def sbody(s, carry):
            cids = [(s * KS + b) * NS + tid for b in range(KS)]
            for b in range(KS):
                @pl.when(cids[b] < NCHUNK)
                def _(b=b):
                    off = cids[b] * C
                    pltpu.async_copy(wmsg.at[pl.ds(off, C)], wrows[b],
                                     semw[b])
            for b in range(KS):
                @pl.when(cids[b] < NCHUNK)
                def _(b=b):
                    off = cids[b] * C
                    pltpu.sync_copy(dst.at[pl.ds(off, C)], dsti[b])
                    for kk in range(C // 16):
                        v = dsti[b][pl.ds(kk * 16, 16)]
                        rel = v - base_n
                        valid = (rel >= 0) & (rel < ZR)
                        idxb[b][pl.ds(kk * 16, 16)] = jnp.where(
                            valid, rel, ZR)
                    pltpu.make_async_copy(wmsg.at[pl.ds(off, C)], wrows[b],
                                          semw[b]).wait()
                    pltpu.sync_copy(wrows[b], table.at[idxb[b]], add=True)

            return carry

        lax.fori_loop(0, SSTEP, sbody, 0)
        plsc.subcore_barrier()

        def wbody(j, carry):
            rcid = j * NS + tid

            @pl.when(rcid < ZR // C)
            def _():
                r = rcid * C
                pltpu.sync_copy(table.at[pl.ds(r, C)], wrows[0])
                pltpu.sync_copy(wrows[0], acc_out.at[pl.ds(base_n + r, C)])

            return carry

        lax.fori_loop(0, WITER, wbody, 0)

        tail = ZR - (ZR // C) * C
        if tail:
            @pl.when(tid == 0)
            def _():
                r = (ZR // C) * C
                pltpu.sync_copy(table.at[pl.ds(r, tail)],
                                wrows[0].at[pl.ds(0, tail)])
                pltpu.sync_copy(wrows[0].at[pl.ds(0, tail)],
                                acc_out.at[pl.ds(base_n + r, tail)])

        plsc.subcore_barrier()


def kernel(node_feature, node_type, edge_index, edge_type, W_emb, b_emb,
           W_ad, b_ad, Wq, bq, Wk, bk, Wv, bv, Wa, ba, rel_att, rel_msg,
           rel_pri, skip, ln_g, ln_b):
    f32 = jnp.float32
    x0 = node_feature[0]
    src = edge_index[0]
    dst = edge_index[1]

    b8 = np.zeros((8, 128), np.float32)
    b4 = np.zeros((8, 64), np.float32)
    for h in range(4):
        b8[h, h * 16:(h + 1) * 16] = 1.0
        b8[h, 64 + h] = 1.0
        b4[h, h * 16:(h + 1) * 16] = 1.0
    b8 = jnp.asarray(b8)
    b4 = jnp.asarray(b4)
    seg = (np.arange(64)[:, None] // 16 == np.arange(8)[None, :]).astype(
        np.float32)
    seg = jnp.asarray(seg)

    def fold(l):
        wkr = jnp.einsum('chd,hdf->chf', Wk[l].reshape(64, H, DK),
                         rel_att[l]).reshape(64, 64)
        bkr = jnp.einsum('hd,hdf->hf', bk[l].reshape(H, DK),
                         rel_att[l]).reshape(1, 64)
        wvr = jnp.einsum('chd,hdf->chf', Wv[l].reshape(64, H, DK),
                         rel_msg[l]).reshape(64, 64)
        bvr = jnp.einsum('hd,hdf->hf', bv[l].reshape(H, DK),
                         rel_msg[l]).reshape(1, 64)
        wkv = jnp.concatenate([wkr, wvr], axis=1)
        bkv = jnp.concatenate([bkr, bvr], axis=1)
        sp = seg * jnp.pad(rel_pri[l], (0, 4))[None, :] * 0.25
        alpha = jax.nn.sigmoid(skip[l])
        return (wkv, bkv, sp, Wa[l] * alpha, (ba[l] * alpha).reshape(1, 64),
                ((1.0 - alpha) * jnp.ones((64,))).reshape(1, 64).astype(f32),
                ln_g[l].reshape(1, 64), ln_b[l].reshape(1, 64))

    wkv0, bkv0, sp0, wa0, ba0, oma0, g0, be0 = fold(0)
    wkv1, bkv1, sp1, wa1, ba1, oma1, g1, be1 = fold(1)

    wq0 = jnp.pad(Wq[0], ((0, 0), (0, 64)))
    wq1 = jnp.pad(Wq[1], ((0, 0), (0, 64)))
    bq0 = jnp.pad(bq[0], (0, 64)).reshape(1, 128)
    bq1 = jnp.pad(bq[1], (0, 64)).reshape(1, 128)
    zin = jnp.zeros((C, 128), f32)

    x, q, kv = _encode_call(x0, W_emb, b_emb.reshape(1, 256), W_ad,
                            b_ad.reshape(1, 64), wq0, bq0, wkv0, bkv0)

    gq, gkv = _gather_kernel(q, kv, src, dst)
    w68 = _edge_call(gq, gkv, sp0, b8)
    acc = _scatter_kernel(w68, dst, zin)[:N]
    x, q, kv = _post_qkv_call(acc, x, b4, wa0, ba0, oma0, g0, be0,
                              wq1, bq1, wkv1, bkv1)

    gq, gkv = _gather_kernel(q, kv, src, dst)
    w68 = _edge_call(gq, gkv, sp1, b8)
    acc = _scatter_kernel(w68, dst, zin)[:N]
    x = _post_call(acc, x, b4, wa1, ba1, oma1, g1, be1)
    return x


# speedup vs baseline: 1.4380x; 1.4380x over previous
import functools

import jax
import jax.numpy as jnp
import numpy as np
from jax import lax
from jax.experimental import pallas as pl
from jax.experimental.pallas import tpu as pltpu
from jax.experimental.pallas import tpu_sc as plsc

N = 50000
E = 800000
HID = 64
H = 4
DK = 16
NC = 2
NS = 16
NW = NC * NS

C = 128
NCHUNK = E // C
GITER = (NCHUNK + NW - 1) // NW
SITER = (NCHUNK + NS - 1) // NS
CS = 64
NCHS = E // CS
KS = 3
SITER_S = (NCHS + NS - 1) // NS
SSTEP = (SITER_S + KS - 1) // KS
NZ = 4
ZR = 12512
NACC = NZ * ZR
TROWS = 12544
ZITER = TROWS // C // NS + 1
WITER = ZR // C // NS + 1


def _encode_body(x0, we, be, wa, ba, wq, bq, wkv, bkv, x_out, q_out, kv_out):
    h = jnp.maximum(x0[...] @ we[...] + be[...], 0.0)
    x = jnp.tanh(h @ wa[...] + ba[...])
    x_out[...] = x
    q_out[...] = x @ wq[...] + bq[...]
    kv_out[...] = x @ wkv[...] + bkv[...]


def _edge_body(gq, gkv, sp, b8, w_out):
    qk = gq[:, :64] * gkv[:, :64]
    att = jnp.minimum(qk @ sp[...], 75.0)
    e8 = jnp.exp(att)
    vext = jnp.concatenate(
        [gkv[:, 64:], jnp.ones((gkv.shape[0], 64), jnp.float32)], axis=1)
    w_out[...] = (e8 @ b8[...]) * vext


def _post_body(acc, x, b4, wa, ba, oma, g, b, x_out):
    rec4 = 1.0 / (acc[:, 64:68] + 1e-16)
    rec8 = jnp.concatenate(
        [rec4, jnp.zeros((acc.shape[0], 4), jnp.float32)], axis=1)
    aggr = acc[:, :64] * (rec8 @ b4[...])
    out = jax.nn.gelu(aggr)
    res = out @ wa[...] + ba[...] + x[...] * oma[...]
    mu = jnp.mean(res, axis=-1, keepdims=True)
    var = jnp.mean((res - mu) ** 2, axis=-1, keepdims=True)
    x_out[...] = (res - mu) / jnp.sqrt(var + 1e-5) * g[...] + b[...]


def _post_qkv_body(acc, x, b4, wa, ba, oma, g, b, wq, bq, wkv, bkv,
                   x_out, q_out, kv_out):
    rec4 = 1.0 / (acc[:, 64:68] + 1e-16)
    rec8 = jnp.concatenate(
        [rec4, jnp.zeros((acc.shape[0], 4), jnp.float32)], axis=1)
    aggr = acc[:, :64] * (rec8 @ b4[...])
    out = jax.nn.gelu(aggr)
    res = out @ wa[...] + ba[...] + x[...] * oma[...]
    mu = jnp.mean(res, axis=-1, keepdims=True)
    var = jnp.mean((res - mu) ** 2, axis=-1, keepdims=True)
    y = (res - mu) / jnp.sqrt(var + 1e-5) * g[...] + b[...]
    x_out[...] = y
    q_out[...] = y @ wq[...] + bq[...]
    kv_out[...] = y @ wkv[...] + bkv[...]


def _row_spec(bs, w):
    return pl.BlockSpec((bs, w), lambda i: (i, 0))


def _full_spec(r, c):
    return pl.BlockSpec((r, c), lambda i: (0, 0))


def _encode_call(x0, we, be, wa, ba, wq, bq, wkv, bkv):
    bs = 2000
    return pl.pallas_call(
        _encode_body,
        grid=(N // bs,),
        in_specs=[_row_spec(bs, 128), _full_spec(128, 256), _full_spec(1, 256),
                  _full_spec(256, 64), _full_spec(1, 64), _full_spec(64, 128),
                  _full_spec(1, 128), _full_spec(64, 128), _full_spec(1, 128)],
        out_specs=(_row_spec(bs, 64), _row_spec(bs, 128), _row_spec(bs, 128)),
        out_shape=(jax.ShapeDtypeStruct((N, 64), jnp.float32),
                   jax.ShapeDtypeStruct((N, 128), jnp.float32),
                   jax.ShapeDtypeStruct((N, 128), jnp.float32)),
    )(x0, we, be, wa, ba, wq, bq, wkv, bkv)


def _edge_call(gq, gkv, sp, b8):
    bs = 4000
    return pl.pallas_call(
        _edge_body,
        grid=(E // bs,),
        in_specs=[_row_spec(bs, 128), _row_spec(bs, 128), _full_spec(64, 8),
                  _full_spec(8, 128)],
        out_specs=_row_spec(bs, 128),
        out_shape=jax.ShapeDtypeStruct((E, 128), jnp.float32),
    )(gq, gkv, sp, b8)


def _post_call(acc, x, b4, wa, ba, oma, g, b):
    bs = 2000
    return pl.pallas_call(
        _post_body,
        grid=(N // bs,),
        in_specs=[_row_spec(bs, 128), _row_spec(bs, 64), _full_spec(8, 64),
                  _full_spec(64, 64), _full_spec(1, 64), _full_spec(1, 64),
                  _full_spec(1, 64), _full_spec(1, 64)],
        out_specs=_row_spec(bs, 64),
        out_shape=jax.ShapeDtypeStruct((N, 64), jnp.float32),
    )(acc, x, b4, wa, ba, oma, g, b)


def _post_qkv_call(acc, x, b4, wa, ba, oma, g, b, wq, bq, wkv, bkv):
    bs = 2000
    return pl.pallas_call(
        _post_qkv_body,
        grid=(N // bs,),
        in_specs=[_row_spec(bs, 128), _row_spec(bs, 64), _full_spec(8, 64),
                  _full_spec(64, 64), _full_spec(1, 64), _full_spec(1, 64),
                  _full_spec(1, 64), _full_spec(1, 64), _full_spec(64, 128),
                  _full_spec(1, 128), _full_spec(64, 128), _full_spec(1, 128)],
        out_specs=(_row_spec(bs, 64), _row_spec(bs, 128), _row_spec(bs, 128)),
        out_shape=(jax.ShapeDtypeStruct((N, 64), jnp.float32),
                   jax.ShapeDtypeStruct((N, 128), jnp.float32),
                   jax.ShapeDtypeStruct((N, 128), jnp.float32)),
    )(acc, x, b4, wa, ba, oma, g, b, wq, bq, wkv, bkv)


_MESH = plsc.VectorSubcoreMesh(
    core_axis_name="c", subcore_axis_name="s", num_cores=NC, num_subcores=NS)


KG = 3
GSTEP = (GITER + KG - 1) // KG


@functools.partial(
    pl.kernel,
    out_type=(jax.ShapeDtypeStruct((E, 128), jnp.float32),
              jax.ShapeDtypeStruct((E, 128), jnp.float32)),
    mesh=_MESH,
    scratch_types=[
        [pltpu.VMEM((C,), jnp.int32)] * KG,
        [pltpu.VMEM((C,), jnp.int32)] * KG,
        [pltpu.VMEM((C, 128), jnp.float32)] * KG,
        [pltpu.VMEM((C, 128), jnp.float32)] * KG,
        [pltpu.SemaphoreType.DMA] * KG,
        [pltpu.SemaphoreType.DMA] * KG,
        [pltpu.SemaphoreType.DMA] * KG,
    ],
)
def _gather_kernel(qtab, kvtab, src, dst, gq_out, gkv_out,
                   dsti, srci, qrows, kvrows, semi, semq, semk):
    wid = lax.axis_index("s") * NC + lax.axis_index("c")

    def body(s, carry):
        cids = [(s * KG + b) * NW + wid for b in range(KG)]
        for b in range(KG):
            @pl.when(cids[b] < NCHUNK)
            def _(b=b):
                off = cids[b] * C
                pltpu.async_copy(dst.at[pl.ds(off, C)], dsti[b], semi[b])
                pltpu.async_copy(src.at[pl.ds(off, C)], srci[b], semi[b])
        for b in range(KG):
            @pl.when(cids[b] < NCHUNK)
            def _(b=b):
                off = cids[b] * C
                pltpu.make_async_copy(dst.at[pl.ds(off, C)], dsti[b],
                                      semi[b]).wait()
                pltpu.make_async_copy(src.at[pl.ds(off, C)], srci[b],
                                      semi[b]).wait()
                pltpu.async_copy(qtab.at[dsti[b]], qrows[b], semq[b])
                pltpu.async_copy(kvtab.at[srci[b]], kvrows[b], semk[b])
        for b in range(KG):
            @pl.when(cids[b] < NCHUNK)
            def _(b=b):
                off = cids[b] * C
                pltpu.make_async_copy(qtab.at[dsti[b]], qrows[b],
                                      semq[b]).wait()
                pltpu.make_async_copy(kvtab.at[srci[b]], kvrows[b],
                                      semk[b]).wait()
                pltpu.sync_copy(qrows[b], gq_out.at[pl.ds(off, C)])
                pltpu.sync_copy(kvrows[b], gkv_out.at[pl.ds(off, C)])

        return carry

    lax.fori_loop(0, GSTEP, body, 0)


@functools.partial(
    pl.kernel,
    out_type=jax.ShapeDtypeStruct((NACC, 128), jnp.float32),
    mesh=_MESH,
    scratch_types=[
        pltpu.VMEM((C,), jnp.int32),
        pltpu.VMEM((C,), jnp.int32),
        pltpu.VMEM((C, 128), jnp.float32),
        pltpu.VMEM_SHARED((TROWS, 128), jnp.float32),
        pltpu.SemaphoreType.DMA,
    ],
)
def _scatter_kernel(wmsg, dst, zin, acc_out, dsti, idxb, wrows, table, semw):
    cidx = lax.axis_index("c")
    tid = lax.axis_index("s")

    for zi in range(NZ // NC):
        zone = cidx * (NZ // NC) + zi
        base_n = zone * ZR

        pltpu.sync_copy(zin, wrows)

        def zbody(j, carry):
            rcid = j * NS + tid

            @pl.when(rcid < TROWS // C)
            def _():
                pltpu.sync_copy(wrows, table.at[pl.ds(rcid * C, C)])

            return carry

        lax.fori_loop(0, ZITER, zbody, 0)
        plsc.subcore_barrier()

        def sbody(i, carry):
            cid = i * NS + tid

            @pl.when(cid < NCHUNK)
            def _():
                off = cid * C
                pltpu.async_copy(wmsg.at[pl.ds(off, C)], wrows, semw)
                pltpu.sync_copy(dst.at[pl.ds(off, C)], dsti)
                for kk in range(C // 16):
                    v = dsti[pl.ds(kk * 16, 16)]
                    rel = v - base_n
                    valid = (rel >= 0) & (rel < ZR)
                    idxb[pl.ds(kk * 16, 16)] = jnp.where(valid, rel, ZR)
                pltpu.make_async_copy(wmsg.at[pl.ds(off, C)], wrows,
                                      semw).wait()
                pltpu.sync_copy(wrows, table.at[idxb], add=True)

            return carry

        lax.fori_loop(0, SITER, sbody, 0)
        plsc.subcore_barrier()

        def wbody(j, carry):
            rcid = j * NS + tid

            @pl.when(rcid < ZR // C)
            def _():
                r = rcid * C
                pltpu.sync_copy(table.at[pl.ds(r, C)], wrows)
                pltpu.sync_copy(wrows, acc_out.at[pl.ds(base_n + r, C)])

            return carry

        lax.fori_loop(0, WITER, wbody, 0)

        tail = ZR - (ZR // C) * C
        if tail:
            @pl.when(tid == 0)
            def _():
                r = (ZR // C) * C
                pltpu.sync_copy(table.at[pl.ds(r, tail)],
                                wrows.at[pl.ds(0, tail)])
                pltpu.sync_copy(wrows.at[pl.ds(0, tail)],
                                acc_out.at[pl.ds(base_n + r, tail)])

        plsc.subcore_barrier()


def kernel(node_feature, node_type, edge_index, edge_type, W_emb, b_emb,
           W_ad, b_ad, Wq, bq, Wk, bk, Wv, bv, Wa, ba, rel_att, rel_msg,
           rel_pri, skip, ln_g, ln_b):
    f32 = jnp.float32
    x0 = node_feature[0]
    src = edge_index[0]
    dst = edge_index[1]

    b8 = np.zeros((8, 128), np.float32)
    b4 = np.zeros((8, 64), np.float32)
    for h in range(4):
        b8[h, h * 16:(h + 1) * 16] = 1.0
        b8[h, 64 + h] = 1.0
        b4[h, h * 16:(h + 1) * 16] = 1.0
    b8 = jnp.asarray(b8)
    b4 = jnp.asarray(b4)
    seg = (np.arange(64)[:, None] // 16 == np.arange(8)[None, :]).astype(
        np.float32)
    seg = jnp.asarray(seg)

    def fold(l):
        wkr = jnp.einsum('chd,hdf->chf', Wk[l].reshape(64, H, DK),
                         rel_att[l]).reshape(64, 64)
        bkr = jnp.einsum('hd,hdf->hf', bk[l].reshape(H, DK),
                         rel_att[l]).reshape(1, 64)
        wvr = jnp.einsum('chd,hdf->chf', Wv[l].reshape(64, H, DK),
                         rel_msg[l]).reshape(64, 64)
        bvr = jnp.einsum('hd,hdf->hf', bv[l].reshape(H, DK),
                         rel_msg[l]).reshape(1, 64)
        wkv = jnp.concatenate([wkr, wvr], axis=1)
        bkv = jnp.concatenate([bkr, bvr], axis=1)
        sp = seg * jnp.pad(rel_pri[l], (0, 4))[None, :] * 0.25
        alpha = jax.nn.sigmoid(skip[l])
        return (wkv, bkv, sp, Wa[l] * alpha, (ba[l] * alpha).reshape(1, 64),
                ((1.0 - alpha) * jnp.ones((64,))).reshape(1, 64).astype(f32),
                ln_g[l].reshape(1, 64), ln_b[l].reshape(1, 64))

    wkv0, bkv0, sp0, wa0, ba0, oma0, g0, be0 = fold(0)
    wkv1, bkv1, sp1, wa1, ba1, oma1, g1, be1 = fold(1)

    wq0 = jnp.pad(Wq[0], ((0, 0), (0, 64)))
    wq1 = jnp.pad(Wq[1], ((0, 0), (0, 64)))
    bq0 = jnp.pad(bq[0], (0, 64)).reshape(1, 128)
    bq1 = jnp.pad(bq[1], (0, 64)).reshape(1, 128)
    zin = jnp.zeros((C, 128), f32)

    x, q, kv = _encode_call(x0, W_emb, b_emb.reshape(1, 256), W_ad,
                            b_ad.reshape(1, 64), wq0, bq0, wkv0, bkv0)

    gq, gkv = _gather_kernel(q, kv, src, dst)
    w68 = _edge_call(gq, gkv, sp0, b8)
    acc = _scatter_kernel(w68, dst, zin)[:N]
    x, q, kv = _post_qkv_call(acc, x, b4, wa0, ba0, oma0, g0, be0,
                              wq1, bq1, wkv1, bkv1)

    gq, gkv = _gather_kernel(q, kv, src, dst)
    w68 = _edge_call(gq, gkv, sp1, b8)
    acc = _scatter_kernel(w68, dst, zin)[:N]
    x = _post_call(acc, x, b4, wa1, ba1, oma1, g1, be1)
    return x
